# baseline (speedup 1.0000x reference)
import functools

import jax
import jax.numpy as jnp
from jax import lax
from jax.experimental import pallas as pl
from jax.experimental.pallas import tpu as pltpu
from jax.experimental.pallas import tpu_sc as plsc

_N = 10000
_E = 320000
_G = 64
_HID = 35
_DP = 48
_NS = 16
_EPT = _E // _NS
_CH = 80
_NCH = _EPT // _CH
_RPT = 624
_RTAIL = _N - _RPT * _NS
_ZR = 48


def _linear_body(x_ref, w_ref, b_ref, o_ref):
    o_ref[0] = (
        jnp.dot(x_ref[0], w_ref[...], preferred_element_type=jnp.float32)
        + b_ref[...]
    )


def _linear(x, w, b, bn):
    s, m, k = x.shape
    dp = w.shape[1]
    return pl.pallas_call(
        _linear_body,
        grid=(s, m // bn),
        in_specs=[
            pl.BlockSpec((1, bn, k), lambda i, j: (i, j, 0)),
            pl.BlockSpec((k, dp), lambda i, j: (0, 0)),
            pl.BlockSpec((1, dp), lambda i, j: (0, 0)),
        ],
        out_specs=pl.BlockSpec((1, bn, dp), lambda i, j: (i, j, 0)),
        out_shape=jax.ShapeDtypeStruct((s, m, dp), jnp.float32),
    )(x, w, b)


def _edge_proj_body(xr_ref, xg_ref, w_ref, b_ref, o_ref):
    i = pl.program_id(0)
    x = jnp.where(i == 0, xr_ref[...], xg_ref[...])
    dn = (((0,), (0,)), ((), ()))
    o_ref[0] = (
        lax.dot_general(x, w_ref[...], dn, preferred_element_type=jnp.float32)
        + b_ref[...]
    )


def _edge_proj(xr_t, xg_t, w, b, bn=2560):
    k, m = xr_t.shape
    dp = w.shape[1]
    tspec = pl.BlockSpec((k, bn), lambda i, j: (0, j))
    return pl.pallas_call(
        _edge_proj_body,
        grid=(2, m // bn),
        in_specs=[
            tspec,
            tspec,
            pl.BlockSpec((k, dp), lambda i, j: (0, 0)),
            pl.BlockSpec((1, dp), lambda i, j: (0, 0)),
        ],
        out_specs=pl.BlockSpec((1, bn, dp), lambda i, j: (i, j, 0)),
        out_shape=jax.ShapeDtypeStruct((2, m, dp), jnp.float32),
    )(xr_t, xg_t, w, b)


def _mlp_body(h_ref, a_ref, w1_ref, b1_ref, w2_ref, b2_ref, o_ref):
    z = h_ref[0] + a_ref[0]
    t = jnp.maximum(
        jnp.dot(z, w1_ref[...], preferred_element_type=jnp.float32) + b1_ref[...],
        0.0,
    )
    o_ref[0] = (
        jnp.dot(t, w2_ref[...], preferred_element_type=jnp.float32) + b2_ref[...]
    )


def _mlp(h, agg, w1, b1, w2, b2, bn=2000):
    s, m, dp = h.shape
    return pl.pallas_call(
        _mlp_body,
        grid=(s, m // bn),
        in_specs=[
            pl.BlockSpec((1, bn, dp), lambda i, j: (i, j, 0)),
            pl.BlockSpec((1, bn, dp), lambda i, j: (i, j, 0)),
            pl.BlockSpec((dp, 64), lambda i, j: (0, 0)),
            pl.BlockSpec((1, 64), lambda i, j: (0, 0)),
            pl.BlockSpec((64, dp), lambda i, j: (0, 0)),
            pl.BlockSpec((1, dp), lambda i, j: (0, 0)),
        ],
        out_specs=pl.BlockSpec((1, bn, dp), lambda i, j: (i, j, 0)),
        out_shape=jax.ShapeDtypeStruct((s, m, dp), jnp.float32),
    )(h, agg, w1, b1, w2, b2)


def _mlp_seg_body(h_ref, a_ref, bat_ref, w1_ref, b1_ref, w2_ref, b2_ref,
                  g1_ref, g2_ref):
    j = pl.program_id(1)
    nf1 = h_ref[0]
    z = nf1 + a_ref[0]
    t = jnp.maximum(
        jnp.dot(z, w1_ref[...], preferred_element_type=jnp.float32) + b1_ref[...],
        0.0,
    )
    nf2 = jnp.dot(t, w2_ref[...], preferred_element_type=jnp.float32) + b2_ref[...]
    bn = nf1.shape[0]
    oh = (bat_ref[0] == lax.broadcasted_iota(jnp.int32, (bn, _G), 1)).astype(
        jnp.float32
    )
    dn = (((0,), (0,)), ((), ()))
    g1c = lax.dot_general(oh, nf1, dn, preferred_element_type=jnp.float32)
    g2c = lax.dot_general(oh, nf2, dn, preferred_element_type=jnp.float32)

    @pl.when(j == 0)
    def _():
        g1_ref[...] = jnp.zeros_like(g1_ref)
        g2_ref[...] = jnp.zeros_like(g2_ref)

    g1_ref[0] += g1c
    g2_ref[0] += g2c


def _mlp_seg(nf1, agg2, bat, w1, b1, w2, b2, bn=2000):
    s, m, dp = nf1.shape
    gspec = pl.BlockSpec((1, _G, dp), lambda i, j: (i, 0, 0))
    return pl.pallas_call(
        _mlp_seg_body,
        grid=(s, m // bn),
        in_specs=[
            pl.BlockSpec((1, bn, dp), lambda i, j: (i, j, 0)),
            pl.BlockSpec((1, bn, dp), lambda i, j: (i, j, 0)),
            pl.BlockSpec((1, bn, 1), lambda i, j: (i, j, 0)),
            pl.BlockSpec((dp, 64), lambda i, j: (0, 0)),
            pl.BlockSpec((1, 64), lambda i, j: (0, 0)),
            pl.BlockSpec((64, dp), lambda i, j: (0, 0)),
            pl.BlockSpec((1, dp), lambda i, j: (0, 0)),
        ],
        out_specs=[gspec, gspec],
        out_shape=[
            jax.ShapeDtypeStruct((s, _G, dp), jnp.float32),
            jax.ShapeDtypeStruct((s, _G, dp), jnp.float32),
        ],
    )(nf1, agg2, bat, w1, b1, w2, b2)


def _mmd_body(g1_ref, g2_ref, o_ref):
    def pair_sum(p, q):
        qa = g1_ref[q]
        qb = g2_ref[q]

        def body(i, acc):
            pa = g1_ref[p, pl.ds(i, 1), :]
            pb = g2_ref[p, pl.ds(i, 1), :]
            s = jnp.sum((pa - qa) ** 2 + (pb - qb) ** 2, axis=1, keepdims=True)
            d = jnp.where(s > 0, jnp.sqrt(jnp.where(s > 0, s, 1.0)), 0.0)
            return acc + jnp.sum(jnp.exp(-0.5 * d))

        return lax.fori_loop(0, _G, body, 0.0)

    sxx = pair_sum(0, 0)
    syy = pair_sum(1, 1)
    sxy = pair_sum(0, 1)
    denom = float((2 * _HID) ** 2)
    o_ref[...] = jnp.broadcast_to((sxx + syy - 2.0 * sxy) / denom, (1, 1))


def _mmd(g1, g2):
    return pl.pallas_call(
        _mmd_body,
        out_shape=jax.ShapeDtypeStruct((1, 1), jnp.float32),
    )(g1, g2)


def _edge_pass(table_flat, ei_flat, ea_flat):
    mesh = plsc.VectorSubcoreMesh(core_axis_name="c", subcore_axis_name="s")

    @functools.partial(
        pl.kernel,
        mesh=mesh,
        out_type=jax.ShapeDtypeStruct((2 * _N, _DP), jnp.float32),
        scratch_types=[
            pltpu.VMEM((2, _CH), jnp.int32),
            pltpu.VMEM((2, _CH), jnp.int32),
            pltpu.VMEM((2, _CH, _DP), jnp.float32),
            pltpu.VMEM((2, _CH, 128), jnp.float32),
            pltpu.VMEM((_ZR, _DP), jnp.float32),
            pltpu.VMEM_SHARED((_N, _DP), jnp.float32),
            pltpu.SemaphoreType.DMA,
            pltpu.SemaphoreType.DMA,
            pltpu.SemaphoreType.DMA,
            pltpu.SemaphoreType.DMA,
        ],
        compiler_params=pltpu.CompilerParams(use_tc_tiling_on_sc=False),
    )
    def ek(tbl_hbm, ei_hbm, ea_hbm, out_hbm,
           src2, dst2, rows2, ea2, zer_v, acc_sh,
           sem_l, sem_d, sem_g, sem_s):
        c = lax.axis_index("c")
        s = lax.axis_index("s")
        zvec = jnp.zeros((16,), jnp.float32)

        def zb(r, carry):
            zer_v[r, pl.ds(0, 16)] = zvec
            zer_v[r, pl.ds(16, 16)] = zvec
            zer_v[r, pl.ds(32, 16)] = zvec
            return carry

        lax.fori_loop(0, _ZR, zb, 0)

        def zc(j, carry):
            pltpu.sync_copy(
                zer_v, acc_sh.at[pl.ds(s * _RPT + j * _ZR, _ZR), :]
            )
            return carry

        lax.fori_loop(0, _RPT // _ZR, zc, 0)

        @pl.when(s == _NS - 1)
        def _():
            pltpu.sync_copy(
                zer_v.at[pl.ds(0, _RTAIL), :],
                acc_sh.at[pl.ds(_RPT * _NS, _RTAIL), :],
            )

        plsc.subcore_barrier()
        row_off = c * _N

        ebase = c * (2 * _E) + s * _EPT
        eabase = c * _E + s * _EPT

        def issue_srcea(i, b):
            pltpu.async_copy(
                ei_hbm.at[pl.ds(ebase + i * _CH, _CH)], src2.at[b], sem_l
            )
            pltpu.async_copy(
                ea_hbm.at[pl.ds(eabase + i * _CH, _CH), :],
                ea2.at[b],
                sem_l,
            )

        def wait_srcea(b):
            pltpu.make_async_copy(
                ei_hbm.at[pl.ds(ebase, _CH)], src2.at[b], sem_l
            ).wait()
            pltpu.make_async_copy(
                ea_hbm.at[pl.ds(eabase, _CH), :], ea2.at[b], sem_l
            ).wait()

        def issue_dst(i, b):
            pltpu.async_copy(
                ei_hbm.at[pl.ds(ebase + _E + i * _CH, _CH)], dst2.at[b], sem_d
            )

        def shift_src(b):
            for k in range(_CH // 16):
                sl = pl.ds(k * 16, 16)
                src2[b, sl] = src2[b, sl] + row_off

        issue_srcea(0, 0)
        issue_srcea(1, 1)
        issue_dst(0, 0)
        wait_srcea(0)
        shift_src(0)
        pltpu.async_copy(tbl_hbm.at[src2.at[0]], rows2.at[0], sem_g)

        def ch(t, carry):
            for b in range(2):
                ob = 1 - b
                i = 2 * t + b
                pltpu.make_async_copy(
                    tbl_hbm.at[src2.at[b]], rows2.at[b], sem_g
                ).wait()

                @pl.when(i >= 1)
                def _():
                    pltpu.make_async_copy(
                        rows2.at[ob], acc_sh.at[dst2.at[ob]], sem_s
                    ).wait()

                @pl.when(i + 1 < _NCH)
                def _():
                    issue_dst(i + 1, ob)
                    wait_srcea(ob)
                    shift_src(ob)
                    pltpu.async_copy(
                        tbl_hbm.at[src2.at[ob]], rows2.at[ob], sem_g
                    )

                def rb(e, inner):
                    for j2 in range(_DP // 16):
                        sl2 = pl.ds(j2 * 16, 16)
                        rows2[b, e, sl2] = jnp.maximum(
                            rows2[b, e, sl2] + ea2[b, e, sl2], 0.0,
                        )
                    return inner

                lax.fori_loop(0, _CH, rb, 0, unroll=4)
                pltpu.make_async_copy(
                    ei_hbm.at[pl.ds(ebase, _CH)], dst2.at[b], sem_d
                ).wait()
                pltpu.async_copy(
                    rows2.at[b], acc_sh.at[dst2.at[b]], sem_s, add=True
                )

                @pl.when(i + 2 < _NCH)
                def _():
                    issue_srcea(i + 2, b)
            return carry

        lax.fori_loop(0, _NCH // 2, ch, 0)
        pltpu.make_async_copy(
            rows2.at[1], acc_sh.at[dst2.at[1]], sem_s
        ).wait()
        plsc.subcore_barrier()
        pltpu.sync_copy(
            acc_sh.at[pl.ds(s * _RPT, _RPT), :],
            out_hbm.at[pl.ds(c * _N + s * _RPT, _RPT), :],
        )

        @pl.when(s == _NS - 1)
        def _():
            pltpu.sync_copy(
                acc_sh.at[pl.ds(_RPT * _NS, _RTAIL), :],
                out_hbm.at[pl.ds(c * _N + _RPT * _NS, _RTAIL), :],
            )

    return ek(table_flat, ei_flat, ea_flat)


def kernel(x_real, edge_index_real, batch_real, edge_attr_real,
           x_gen, edge_index_gen, batch_gen, edge_attr_gen,
           W_node, b_node, W_edge, b_edge, W1, b1, W2, b2):
    pad = _DP - _HID
    wn = jnp.pad(W_node, ((0, 0), (0, pad)))
    bn = jnp.pad(b_node, (0, pad)).reshape(1, _DP)
    we = jnp.pad(W_edge, ((0, 0), (0, 128 - _HID)))
    be = jnp.pad(b_edge, (0, 128 - _HID)).reshape(1, 128)
    w1 = jnp.pad(W1, ((0, pad), (0, 0)))
    b1r = b1.reshape(1, 64)
    w2 = jnp.pad(W2, ((0, 0), (0, pad)))
    b2r = jnp.pad(b2, (0, pad)).reshape(1, _DP)

    x2 = jnp.stack([x_real, x_gen])
    ei_flat = jnp.stack([edge_index_real, edge_index_gen]).reshape(4 * _E)
    bat2 = jnp.stack([batch_real, batch_gen]).reshape(2, _N, 1)

    h2 = _linear(x2, wn, bn, 2000)
    ea_flat = _edge_proj(edge_attr_real.T, edge_attr_gen.T, we, be).reshape(
        2 * _E, 128
    )

    agg1 = _edge_pass(h2.reshape(2 * _N, _DP), ei_flat, ea_flat)
    nf1 = _mlp(h2, agg1.reshape(2, _N, _DP), w1, b1r, w2, b2r)
    agg2 = _edge_pass(nf1.reshape(2 * _N, _DP), ei_flat, ea_flat)
    g1, g2 = _mlp_seg(nf1, agg2.reshape(2, _N, _DP), bat2, w1, b1r, w2, b2r)
    out = _mmd(g1, g2)
    return out[0, 0]


# speedup vs baseline: 1.4216x; 1.4216x over previous
import functools

import jax
import jax.numpy as jnp
from jax import lax
from jax.experimental import pallas as pl
from jax.experimental.pallas import tpu as pltpu
from jax.experimental.pallas import tpu_sc as plsc

_N = 10000
_E = 320000
_G = 64
_HID = 35
_DP = 48
_NS = 16
_EPT = _E // _NS
_CH = 80
_NCH = _EPT // _CH
_RPT = 624
_RTAIL = _N - _RPT * _NS
_ZR = 48


def _linear_body(x_ref, w_ref, b_ref, o_ref):
    o_ref[0] = (
        jnp.dot(x_ref[0], w_ref[...], preferred_element_type=jnp.float32)
        + b_ref[...]
    )


def _linear(x, w, b, bn):
    s, m, k = x.shape
    dp = w.shape[1]
    return pl.pallas_call(
        _linear_body,
        grid=(s, m // bn),
        in_specs=[
            pl.BlockSpec((1, bn, k), lambda i, j: (i, j, 0)),
            pl.BlockSpec((k, dp), lambda i, j: (0, 0)),
            pl.BlockSpec((1, dp), lambda i, j: (0, 0)),
        ],
        out_specs=pl.BlockSpec((1, bn, dp), lambda i, j: (i, j, 0)),
        out_shape=jax.ShapeDtypeStruct((s, m, dp), jnp.float32),
    )(x, w, b)


def _edge_proj_body(xr_ref, xg_ref, w_ref, b_ref, o_ref):
    i = pl.program_id(0)
    x = jnp.where(i == 0, xr_ref[...], xg_ref[...])
    dn = (((0,), (0,)), ((), ()))
    o_ref[0] = (
        lax.dot_general(x, w_ref[...], dn, preferred_element_type=jnp.float32)
        + b_ref[...]
    )


def _edge_proj(xr_t, xg_t, w, b, bn=2560):
    k, m = xr_t.shape
    dp = w.shape[1]
    tspec = pl.BlockSpec((k, bn), lambda i, j: (0, j))
    return pl.pallas_call(
        _edge_proj_body,
        grid=(2, m // bn),
        in_specs=[
            tspec,
            tspec,
            pl.BlockSpec((k, dp), lambda i, j: (0, 0)),
            pl.BlockSpec((1, dp), lambda i, j: (0, 0)),
        ],
        out_specs=pl.BlockSpec((1, bn, dp), lambda i, j: (i, j, 0)),
        out_shape=jax.ShapeDtypeStruct((2, m, dp), jnp.float32),
    )(xr_t, xg_t, w, b)


def _mlp_body(h_ref, a_ref, w1_ref, b1_ref, w2_ref, b2_ref, o_ref):
    z = h_ref[0] + a_ref[0]
    t = jnp.maximum(
        jnp.dot(z, w1_ref[...], preferred_element_type=jnp.float32) + b1_ref[...],
        0.0,
    )
    o_ref[0] = (
        jnp.dot(t, w2_ref[...], preferred_element_type=jnp.float32) + b2_ref[...]
    )


def _mlp(h, agg, w1, b1, w2, b2, bn=2000):
    s, m, dp = h.shape
    return pl.pallas_call(
        _mlp_body,
        grid=(s, m // bn),
        in_specs=[
            pl.BlockSpec((1, bn, dp), lambda i, j: (i, j, 0)),
            pl.BlockSpec((1, bn, dp), lambda i, j: (i, j, 0)),
            pl.BlockSpec((dp, 64), lambda i, j: (0, 0)),
            pl.BlockSpec((1, 64), lambda i, j: (0, 0)),
            pl.BlockSpec((64, dp), lambda i, j: (0, 0)),
            pl.BlockSpec((1, dp), lambda i, j: (0, 0)),
        ],
        out_specs=pl.BlockSpec((1, bn, dp), lambda i, j: (i, j, 0)),
        out_shape=jax.ShapeDtypeStruct((s, m, dp), jnp.float32),
    )(h, agg, w1, b1, w2, b2)


def _mlp_seg_body(h_ref, a_ref, bat_ref, w1_ref, b1_ref, w2_ref, b2_ref,
                  g1_ref, g2_ref):
    j = pl.program_id(1)
    nf1 = h_ref[0]
    z = nf1 + a_ref[0]
    t = jnp.maximum(
        jnp.dot(z, w1_ref[...], preferred_element_type=jnp.float32) + b1_ref[...],
        0.0,
    )
    nf2 = jnp.dot(t, w2_ref[...], preferred_element_type=jnp.float32) + b2_ref[...]
    bn = nf1.shape[0]
    oh = (bat_ref[0] == lax.broadcasted_iota(jnp.int32, (bn, _G), 1)).astype(
        jnp.float32
    )
    dn = (((0,), (0,)), ((), ()))
    g1c = lax.dot_general(oh, nf1, dn, preferred_element_type=jnp.float32)
    g2c = lax.dot_general(oh, nf2, dn, preferred_element_type=jnp.float32)

    @pl.when(j == 0)
    def _():
        g1_ref[...] = jnp.zeros_like(g1_ref)
        g2_ref[...] = jnp.zeros_like(g2_ref)

    g1_ref[0] += g1c
    g2_ref[0] += g2c


def _mlp_seg(nf1, agg2, bat, w1, b1, w2, b2, bn=2000):
    s, m, dp = nf1.shape
    gspec = pl.BlockSpec((1, _G, dp), lambda i, j: (i, 0, 0))
    return pl.pallas_call(
        _mlp_seg_body,
        grid=(s, m // bn),
        in_specs=[
            pl.BlockSpec((1, bn, dp), lambda i, j: (i, j, 0)),
            pl.BlockSpec((1, bn, dp), lambda i, j: (i, j, 0)),
            pl.BlockSpec((1, bn, 1), lambda i, j: (i, j, 0)),
            pl.BlockSpec((dp, 64), lambda i, j: (0, 0)),
            pl.BlockSpec((1, 64), lambda i, j: (0, 0)),
            pl.BlockSpec((64, dp), lambda i, j: (0, 0)),
            pl.BlockSpec((1, dp), lambda i, j: (0, 0)),
        ],
        out_specs=[gspec, gspec],
        out_shape=[
            jax.ShapeDtypeStruct((s, _G, dp), jnp.float32),
            jax.ShapeDtypeStruct((s, _G, dp), jnp.float32),
        ],
    )(nf1, agg2, bat, w1, b1, w2, b2)


def _mmd_body(g1_ref, g2_ref, o_ref):
    def pair_sum(p, q):
        qa = g1_ref[q]
        qb = g2_ref[q]

        def body(i, acc):
            pa = g1_ref[p, pl.ds(i, 1), :]
            pb = g2_ref[p, pl.ds(i, 1), :]
            s = jnp.sum((pa - qa) ** 2 + (pb - qb) ** 2, axis=1, keepdims=True)
            d = jnp.where(s > 0, jnp.sqrt(jnp.where(s > 0, s, 1.0)), 0.0)
            return acc + jnp.sum(jnp.exp(-0.5 * d))

        return lax.fori_loop(0, _G, body, 0.0)

    sxx = pair_sum(0, 0)
    syy = pair_sum(1, 1)
    sxy = pair_sum(0, 1)
    denom = float((2 * _HID) ** 2)
    o_ref[...] = jnp.broadcast_to((sxx + syy - 2.0 * sxy) / denom, (1, 1))


def _mmd(g1, g2):
    return pl.pallas_call(
        _mmd_body,
        out_shape=jax.ShapeDtypeStruct((1, 1), jnp.float32),
    )(g1, g2)


def _edge_pass(table_flat, ei_flat, ea_flat):
    mesh = plsc.VectorSubcoreMesh(core_axis_name="c", subcore_axis_name="s")

    @functools.partial(
        pl.kernel,
        mesh=mesh,
        out_type=jax.ShapeDtypeStruct((2 * _N, _DP), jnp.float32),
        scratch_types=[
            pltpu.VMEM((2, _CH), jnp.int32),
            pltpu.VMEM((2, _CH), jnp.int32),
            pltpu.VMEM((2, _CH, _DP), jnp.float32),
            pltpu.VMEM((2, _CH, 128), jnp.float32),
            pltpu.VMEM((_ZR, _DP), jnp.float32),
            pltpu.VMEM_SHARED((_N, _DP), jnp.float32),
            pltpu.SemaphoreType.DMA,
            pltpu.SemaphoreType.DMA,
            pltpu.SemaphoreType.DMA,
            pltpu.SemaphoreType.DMA,
        ],
        compiler_params=pltpu.CompilerParams(use_tc_tiling_on_sc=False),
    )
    def ek(tbl_hbm, ei_hbm, ea_hbm, out_hbm,
           src2, dst2, rows2, ea2, zer_v, acc_sh,
           sem_l, sem_d, sem_g, sem_s):
        c = lax.axis_index("c")
        s = lax.axis_index("s")
        zvec = jnp.zeros((16,), jnp.float32)

        def zb(r, carry):
            zer_v[r, pl.ds(0, 16)] = zvec
            zer_v[r, pl.ds(16, 16)] = zvec
            zer_v[r, pl.ds(32, 16)] = zvec
            return carry

        lax.fori_loop(0, _ZR, zb, 0)

        def zc(j, carry):
            pltpu.sync_copy(
                zer_v, acc_sh.at[pl.ds(s * _RPT + j * _ZR, _ZR), :]
            )
            return carry

        lax.fori_loop(0, _RPT // _ZR, zc, 0)

        @pl.when(s == _NS - 1)
        def _():
            pltpu.sync_copy(
                zer_v.at[pl.ds(0, _RTAIL), :],
                acc_sh.at[pl.ds(_RPT * _NS, _RTAIL), :],
            )

        plsc.subcore_barrier()
        row_off = c * _N

        ebase = c * (2 * _E) + s * _EPT
        eabase = c * _E + s * _EPT

        def issue_srcea(i, b):
            pltpu.async_copy(
                ei_hbm.at[pl.ds(ebase + i * _CH, _CH)], src2.at[b], sem_l
            )
            pltpu.async_copy(
                ea_hbm.at[pl.ds(eabase + i * _CH, _CH), :],
                ea2.at[b],
                sem_l,
            )

        def wait_srcea(b):
            pltpu.make_async_copy(
                ei_hbm.at[pl.ds(ebase, _CH)], src2.at[b], sem_l
            ).wait()
            pltpu.make_async_copy(
                ea_hbm.at[pl.ds(eabase, _CH), :], ea2.at[b], sem_l
            ).wait()

        def issue_dst(i, b):
            pltpu.async_copy(
                ei_hbm.at[pl.ds(ebase + _E + i * _CH, _CH)], dst2.at[b], sem_d
            )

        def shift_src(b):
            for k in range(_CH // 16):
                sl = pl.ds(k * 16, 16)
                src2[b, sl] = src2[b, sl] + row_off

        issue_srcea(0, 0)
        issue_srcea(1, 1)
        issue_dst(0, 0)
        wait_srcea(0)
        shift_src(0)
        pltpu.async_copy(tbl_hbm.at[src2.at[0]], rows2.at[0], sem_g)

        def ch(t, carry):
            for b in range(2):
                ob = 1 - b
                i = 2 * t + b
                pltpu.make_async_copy(
                    tbl_hbm.at[src2.at[b]], rows2.at[b], sem_g
                ).wait()

                @pl.when(i >= 1)
                def _():
                    pltpu.make_async_copy(
                        rows2.at[ob], acc_sh.at[dst2.at[ob]], sem_s
                    ).wait()

                @pl.when(i + 1 < _NCH)
                def _():
                    issue_dst(i + 1, ob)
                    wait_srcea(ob)
                    shift_src(ob)
                    pltpu.async_copy(
                        tbl_hbm.at[src2.at[ob]], rows2.at[ob], sem_g
                    )

                @plsc.parallel_loop(0, _CH, unroll=4)
                def _(e):
                    for j2 in range(_DP // 16):
                        sl2 = pl.ds(j2 * 16, 16)
                        rows2[b, e, sl2] = jnp.maximum(
                            rows2[b, e, sl2] + ea2[b, e, sl2], 0.0,
                        )
                pltpu.make_async_copy(
                    ei_hbm.at[pl.ds(ebase, _CH)], dst2.at[b], sem_d
                ).wait()
                pltpu.async_copy(
                    rows2.at[b], acc_sh.at[dst2.at[b]], sem_s, add=True
                )

                @pl.when(i + 2 < _NCH)
                def _():
                    issue_srcea(i + 2, b)
            return carry

        lax.fori_loop(0, _NCH // 2, ch, 0)
        pltpu.make_async_copy(
            rows2.at[1], acc_sh.at[dst2.at[1]], sem_s
        ).wait()
        plsc.subcore_barrier()
        pltpu.sync_copy(
            acc_sh.at[pl.ds(s * _RPT, _RPT), :],
            out_hbm.at[pl.ds(c * _N + s * _RPT, _RPT), :],
        )

        @pl.when(s == _NS - 1)
        def _():
            pltpu.sync_copy(
                acc_sh.at[pl.ds(_RPT * _NS, _RTAIL), :],
                out_hbm.at[pl.ds(c * _N + _RPT * _NS, _RTAIL), :],
            )

    return ek(table_flat, ei_flat, ea_flat)


def kernel(x_real, edge_index_real, batch_real, edge_attr_real,
           x_gen, edge_index_gen, batch_gen, edge_attr_gen,
           W_node, b_node, W_edge, b_edge, W1, b1, W2, b2):
    pad = _DP - _HID
    wn = jnp.pad(W_node, ((0, 0), (0, pad)))
    bn = jnp.pad(b_node, (0, pad)).reshape(1, _DP)
    we = jnp.pad(W_edge, ((0, 0), (0, 128 - _HID)))
    be = jnp.pad(b_edge, (0, 128 - _HID)).reshape(1, 128)
    w1 = jnp.pad(W1, ((0, pad), (0, 0)))
    b1r = b1.reshape(1, 64)
    w2 = jnp.pad(W2, ((0, 0), (0, pad)))
    b2r = jnp.pad(b2, (0, pad)).reshape(1, _DP)

    x2 = jnp.stack([x_real, x_gen])
    ei_flat = jnp.stack([edge_index_real, edge_index_gen]).reshape(4 * _E)
    bat2 = jnp.stack([batch_real, batch_gen]).reshape(2, _N, 1)

    h2 = _linear(x2, wn, bn, 2000)
    ea_flat = _edge_proj(edge_attr_real.T, edge_attr_gen.T, we, be).reshape(
        2 * _E, 128
    )

    agg1 = _edge_pass(h2.reshape(2 * _N, _DP), ei_flat, ea_flat)
    nf1 = _mlp(h2, agg1.reshape(2, _N, _DP), w1, b1r, w2, b2r)
    agg2 = _edge_pass(nf1.reshape(2 * _N, _DP), ei_flat, ea_flat)
    g1, g2 = _mlp_seg(nf1, agg2.reshape(2, _N, _DP), bat2, w1, b1r, w2, b2r)
    out = _mmd(g1, g2)
    return out[0, 0]


# speedup vs baseline: 1.4765x; 1.0386x over previous
import functools

import jax
import jax.numpy as jnp
from jax import lax
from jax.experimental import pallas as pl
from jax.experimental.pallas import tpu as pltpu
from jax.experimental.pallas import tpu_sc as plsc

_N = 10000
_E = 320000
_G = 64
_HID = 35
_DP = 48
_NS = 16
_EPT = _E // _NS
_CH = 80
_NCH = _EPT // _CH
_RPT = 624
_RTAIL = _N - _RPT * _NS
_ZR = 48


def _linear_body(x_ref, w_ref, b_ref, o_ref):
    o_ref[0] = (
        jnp.dot(x_ref[0], w_ref[...], preferred_element_type=jnp.float32)
        + b_ref[...]
    )


def _linear(x, w, b, bn):
    s, m, k = x.shape
    dp = w.shape[1]
    return pl.pallas_call(
        _linear_body,
        grid=(s, m // bn),
        in_specs=[
            pl.BlockSpec((1, bn, k), lambda i, j: (i, j, 0)),
            pl.BlockSpec((k, dp), lambda i, j: (0, 0)),
            pl.BlockSpec((1, dp), lambda i, j: (0, 0)),
        ],
        out_specs=pl.BlockSpec((1, bn, dp), lambda i, j: (i, j, 0)),
        out_shape=jax.ShapeDtypeStruct((s, m, dp), jnp.float32),
    )(x, w, b)


def _edge_proj_body(xr_ref, xg_ref, w_ref, b_ref, o_ref):
    i = pl.program_id(0)
    x = jnp.where(i == 0, xr_ref[...], xg_ref[...])
    dn = (((0,), (0,)), ((), ()))
    o_ref[0] = (
        lax.dot_general(x, w_ref[...], dn, preferred_element_type=jnp.float32)
        + b_ref[...]
    )


def _edge_proj(xr_t, xg_t, w, b, bn=2560):
    k, m = xr_t.shape
    dp = w.shape[1]
    tspec = pl.BlockSpec((k, bn), lambda i, j: (0, j))
    return pl.pallas_call(
        _edge_proj_body,
        grid=(2, m // bn),
        in_specs=[
            tspec,
            tspec,
            pl.BlockSpec((k, dp), lambda i, j: (0, 0)),
            pl.BlockSpec((1, dp), lambda i, j: (0, 0)),
        ],
        out_specs=pl.BlockSpec((1, bn, dp), lambda i, j: (i, j, 0)),
        out_shape=jax.ShapeDtypeStruct((2, m, dp), jnp.float32),
    )(xr_t, xg_t, w, b)


def _mlp_body(h_ref, a_ref, w1_ref, b1_ref, w2_ref, b2_ref, o_ref):
    z = h_ref[0] + a_ref[0]
    t = jnp.maximum(
        jnp.dot(z, w1_ref[...], preferred_element_type=jnp.float32) + b1_ref[...],
        0.0,
    )
    o_ref[0] = (
        jnp.dot(t, w2_ref[...], preferred_element_type=jnp.float32) + b2_ref[...]
    )


def _mlp(h, agg, w1, b1, w2, b2, bn=2000):
    s, m, dp = h.shape
    return pl.pallas_call(
        _mlp_body,
        grid=(s, m // bn),
        in_specs=[
            pl.BlockSpec((1, bn, dp), lambda i, j: (i, j, 0)),
            pl.BlockSpec((1, bn, dp), lambda i, j: (i, j, 0)),
            pl.BlockSpec((dp, 64), lambda i, j: (0, 0)),
            pl.BlockSpec((1, 64), lambda i, j: (0, 0)),
            pl.BlockSpec((64, dp), lambda i, j: (0, 0)),
            pl.BlockSpec((1, dp), lambda i, j: (0, 0)),
        ],
        out_specs=pl.BlockSpec((1, bn, dp), lambda i, j: (i, j, 0)),
        out_shape=jax.ShapeDtypeStruct((s, m, dp), jnp.float32),
    )(h, agg, w1, b1, w2, b2)


def _mlp_seg_body(h_ref, a_ref, bat_ref, w1_ref, b1_ref, w2_ref, b2_ref,
                  g1_ref, g2_ref):
    j = pl.program_id(1)
    nf1 = h_ref[0]
    z = nf1 + a_ref[0]
    t = jnp.maximum(
        jnp.dot(z, w1_ref[...], preferred_element_type=jnp.float32) + b1_ref[...],
        0.0,
    )
    nf2 = jnp.dot(t, w2_ref[...], preferred_element_type=jnp.float32) + b2_ref[...]
    bn = nf1.shape[0]
    oh = (bat_ref[0] == lax.broadcasted_iota(jnp.int32, (bn, _G), 1)).astype(
        jnp.float32
    )
    dn = (((0,), (0,)), ((), ()))
    g1c = lax.dot_general(oh, nf1, dn, preferred_element_type=jnp.float32)
    g2c = lax.dot_general(oh, nf2, dn, preferred_element_type=jnp.float32)

    @pl.when(j == 0)
    def _():
        g1_ref[...] = jnp.zeros_like(g1_ref)
        g2_ref[...] = jnp.zeros_like(g2_ref)

    g1_ref[0] += g1c
    g2_ref[0] += g2c


def _mlp_seg(nf1, agg2, bat, w1, b1, w2, b2, bn=2000):
    s, m, dp = nf1.shape
    gspec = pl.BlockSpec((1, _G, dp), lambda i, j: (i, 0, 0))
    return pl.pallas_call(
        _mlp_seg_body,
        grid=(s, m // bn),
        in_specs=[
            pl.BlockSpec((1, bn, dp), lambda i, j: (i, j, 0)),
            pl.BlockSpec((1, bn, dp), lambda i, j: (i, j, 0)),
            pl.BlockSpec((1, bn, 1), lambda i, j: (i, j, 0)),
            pl.BlockSpec((dp, 64), lambda i, j: (0, 0)),
            pl.BlockSpec((1, 64), lambda i, j: (0, 0)),
            pl.BlockSpec((64, dp), lambda i, j: (0, 0)),
            pl.BlockSpec((1, dp), lambda i, j: (0, 0)),
        ],
        out_specs=[gspec, gspec],
        out_shape=[
            jax.ShapeDtypeStruct((s, _G, dp), jnp.float32),
            jax.ShapeDtypeStruct((s, _G, dp), jnp.float32),
        ],
    )(nf1, agg2, bat, w1, b1, w2, b2)


def _mmd_body(g1_ref, g2_ref, o_ref):
    off_diag = lax.broadcasted_iota(
        jnp.int32, (_G, _G), 0
    ) != lax.broadcasted_iota(jnp.int32, (_G, _G), 1)

    def pair_sum(p, q):
        pa, pb = g1_ref[p], g2_ref[p]
        qa, qb = g1_ref[q], g2_ref[q]
        dn_c1 = (((1,), (1,)), ((), ()))
        hi = lax.Precision.HIGHEST
        a2 = jnp.sum(pa * pa + pb * pb, axis=1, keepdims=True)
        b2 = lax.dot_general(
            jnp.ones((1, _DP), jnp.float32), qa * qa + qb * qb,
            dn_c1, preferred_element_type=jnp.float32, precision=hi,
        )
        cross = lax.dot_general(
            pa, qa, dn_c1, preferred_element_type=jnp.float32, precision=hi
        ) + lax.dot_general(
            pb, qb, dn_c1, preferred_element_type=jnp.float32, precision=hi
        )
        d2 = jnp.maximum(a2 + b2 - 2.0 * cross, 0.0)
        if p == q:
            d2 = jnp.where(off_diag, d2, 0.0)
        r = jnp.where(
            d2 > 0,
            jnp.exp(-0.5 * jnp.sqrt(jnp.where(d2 > 0, d2, 1.0))),
            1.0,
        )
        return jnp.sum(r)

    sxx = pair_sum(0, 0)
    syy = pair_sum(1, 1)
    sxy = pair_sum(0, 1)
    denom = float((2 * _HID) ** 2)
    o_ref[...] = jnp.broadcast_to((sxx + syy - 2.0 * sxy) / denom, (1, 1))


def _mmd(g1, g2):
    return pl.pallas_call(
        _mmd_body,
        out_shape=jax.ShapeDtypeStruct((1, 1), jnp.float32),
    )(g1, g2)


def _edge_pass(table_flat, ei_flat, ea_flat):
    mesh = plsc.VectorSubcoreMesh(core_axis_name="c", subcore_axis_name="s")

    @functools.partial(
        pl.kernel,
        mesh=mesh,
        out_type=jax.ShapeDtypeStruct((2 * _N, _DP), jnp.float32),
        scratch_types=[
            pltpu.VMEM((2, _CH), jnp.int32),
            pltpu.VMEM((2, _CH), jnp.int32),
            pltpu.VMEM((2, _CH, _DP), jnp.float32),
            pltpu.VMEM((2, _CH, 128), jnp.float32),
            pltpu.VMEM((_ZR, _DP), jnp.float32),
            pltpu.VMEM_SHARED((_N, _DP), jnp.float32),
            pltpu.SemaphoreType.DMA,
            pltpu.SemaphoreType.DMA,
            pltpu.SemaphoreType.DMA,
            pltpu.SemaphoreType.DMA,
        ],
        compiler_params=pltpu.CompilerParams(use_tc_tiling_on_sc=False),
    )
    def ek(tbl_hbm, ei_hbm, ea_hbm, out_hbm,
           src2, dst2, rows2, ea2, zer_v, acc_sh,
           sem_l, sem_d, sem_g, sem_s):
        c = lax.axis_index("c")
        s = lax.axis_index("s")
        zvec = jnp.zeros((16,), jnp.float32)

        def zb(r, carry):
            zer_v[r, pl.ds(0, 16)] = zvec
            zer_v[r, pl.ds(16, 16)] = zvec
            zer_v[r, pl.ds(32, 16)] = zvec
            return carry

        lax.fori_loop(0, _ZR, zb, 0)

        def zc(j, carry):
            pltpu.sync_copy(
                zer_v, acc_sh.at[pl.ds(s * _RPT + j * _ZR, _ZR), :]
            )
            return carry

        lax.fori_loop(0, _RPT // _ZR, zc, 0)

        @pl.when(s == _NS - 1)
        def _():
            pltpu.sync_copy(
                zer_v.at[pl.ds(0, _RTAIL), :],
                acc_sh.at[pl.ds(_RPT * _NS, _RTAIL), :],
            )

        plsc.subcore_barrier()
        row_off = c * _N

        ebase = c * (2 * _E) + s * _EPT
        eabase = c * _E + s * _EPT

        def issue_srcea(i, b):
            pltpu.async_copy(
                ei_hbm.at[pl.ds(ebase + i * _CH, _CH)], src2.at[b], sem_l
            )
            pltpu.async_copy(
                ea_hbm.at[pl.ds(eabase + i * _CH, _CH), :],
                ea2.at[b],
                sem_l,
            )

        def wait_srcea(b):
            pltpu.make_async_copy(
                ei_hbm.at[pl.ds(ebase, _CH)], src2.at[b], sem_l
            ).wait()
            pltpu.make_async_copy(
                ea_hbm.at[pl.ds(eabase, _CH), :], ea2.at[b], sem_l
            ).wait()

        def issue_dst(i, b):
            pltpu.async_copy(
                ei_hbm.at[pl.ds(ebase + _E + i * _CH, _CH)], dst2.at[b], sem_d
            )

        def shift_src(b):
            for k in range(_CH // 16):
                sl = pl.ds(k * 16, 16)
                src2[b, sl] = src2[b, sl] + row_off

        issue_srcea(0, 0)
        issue_srcea(1, 1)
        issue_dst(0, 0)
        wait_srcea(0)
        shift_src(0)
        pltpu.async_copy(tbl_hbm.at[src2.at[0]], rows2.at[0], sem_g)

        def ch(t, carry):
            for b in range(2):
                ob = 1 - b
                i = 2 * t + b
                pltpu.make_async_copy(
                    tbl_hbm.at[src2.at[b]], rows2.at[b], sem_g
                ).wait()

                @pl.when(i >= 1)
                def _():
                    pltpu.make_async_copy(
                        rows2.at[ob], acc_sh.at[dst2.at[ob]], sem_s
                    ).wait()

                @pl.when(i + 1 < _NCH)
                def _():
                    issue_dst(i + 1, ob)
                    wait_srcea(ob)
                    shift_src(ob)
                    pltpu.async_copy(
                        tbl_hbm.at[src2.at[ob]], rows2.at[ob], sem_g
                    )

                @plsc.parallel_loop(0, _CH, unroll=4)
                def _(e):
                    for j2 in range(_DP // 16):
                        sl2 = pl.ds(j2 * 16, 16)
                        rows2[b, e, sl2] = jnp.maximum(
                            rows2[b, e, sl2] + ea2[b, e, sl2], 0.0,
                        )
                pltpu.make_async_copy(
                    ei_hbm.at[pl.ds(ebase, _CH)], dst2.at[b], sem_d
                ).wait()
                pltpu.async_copy(
                    rows2.at[b], acc_sh.at[dst2.at[b]], sem_s, add=True
                )

                @pl.when(i + 2 < _NCH)
                def _():
                    issue_srcea(i + 2, b)
            return carry

        lax.fori_loop(0, _NCH // 2, ch, 0)
        pltpu.make_async_copy(
            rows2.at[1], acc_sh.at[dst2.at[1]], sem_s
        ).wait()
        plsc.subcore_barrier()
        pltpu.sync_copy(
            acc_sh.at[pl.ds(s * _RPT, _RPT), :],
            out_hbm.at[pl.ds(c * _N + s * _RPT, _RPT), :],
        )

        @pl.when(s == _NS - 1)
        def _():
            pltpu.sync_copy(
                acc_sh.at[pl.ds(_RPT * _NS, _RTAIL), :],
                out_hbm.at[pl.ds(c * _N + _RPT * _NS, _RTAIL), :],
            )

    return ek(table_flat, ei_flat, ea_flat)


def kernel(x_real, edge_index_real, batch_real, edge_attr_real,
           x_gen, edge_index_gen, batch_gen, edge_attr_gen,
           W_node, b_node, W_edge, b_edge, W1, b1, W2, b2):
    pad = _DP - _HID
    wn = jnp.pad(W_node, ((0, 0), (0, pad)))
    bn = jnp.pad(b_node, (0, pad)).reshape(1, _DP)
    we = jnp.pad(W_edge, ((0, 0), (0, 128 - _HID)))
    be = jnp.pad(b_edge, (0, 128 - _HID)).reshape(1, 128)
    w1 = jnp.pad(W1, ((0, pad), (0, 0)))
    b1r = b1.reshape(1, 64)
    w2 = jnp.pad(W2, ((0, 0), (0, pad)))
    b2r = jnp.pad(b2, (0, pad)).reshape(1, _DP)

    x2 = jnp.stack([x_real, x_gen])
    ei_flat = jnp.stack([edge_index_real, edge_index_gen]).reshape(4 * _E)
    bat2 = jnp.stack([batch_real, batch_gen]).reshape(2, _N, 1)

    h2 = _linear(x2, wn, bn, 2000)
    ea_flat = _edge_proj(edge_attr_real.T, edge_attr_gen.T, we, be).reshape(
        2 * _E, 128
    )

    agg1 = _edge_pass(h2.reshape(2 * _N, _DP), ei_flat, ea_flat)
    nf1 = _mlp(h2, agg1.reshape(2, _N, _DP), w1, b1r, w2, b2r)
    agg2 = _edge_pass(nf1.reshape(2 * _N, _DP), ei_flat, ea_flat)
    g1, g2 = _mlp_seg(nf1, agg2.reshape(2, _N, _DP), bat2, w1, b1r, w2, b2r)
    out = _mmd(g1, g2)
    return out[0, 0]
